# baseline (speedup 1.0000x reference)
import functools

import jax
import jax.numpy as jnp
from jax import lax
from jax.experimental import pallas as pl
from jax.experimental.pallas import tpu as pltpu
from jax.experimental.pallas import tpu_sc as plsc

N_NODES = 10000
D = 128
DW = D // 2
E = 320000
NUM_CORES = 2
NUM_SUBCORES = 16
NW = NUM_CORES * NUM_SUBCORES
E_PER_W = E // NW
CHUNK = 80
NCHUNK = E_PER_W // CHUNK
NRING = 3


def _prep_body(x_ref, e_ref, h_ref, s_ref, d_ref):
    xv = x_ref[...]
    ss = jnp.sum(xv * xv, axis=-1, keepdims=True)
    nrm = jnp.sqrt(ss)
    h = xv / jnp.maximum(nrm, 1e-12)
    lo = lax.bitcast_convert_type(
        h[:, :DW].astype(jnp.bfloat16), jnp.uint16).astype(jnp.int32)
    hi = lax.bitcast_convert_type(
        h[:, DW:].astype(jnp.bfloat16), jnp.uint16).astype(jnp.int32)
    h_ref[...] = (hi << 16) | lo
    s_ref[...] = e_ref[0, :]
    d_ref[...] = e_ref[1, :]


def _prep(x, edges):
    return pl.pallas_call(
        _prep_body,
        out_shape=[
            jax.ShapeDtypeStruct((N_NODES, DW), jnp.int32),
            jax.ShapeDtypeStruct((E,), jnp.int32),
            jax.ShapeDtypeStruct((E,), jnp.int32),
        ],
    )(x, edges)


def _sc_edge_dot(hp, src, dst):
    mesh = plsc.VectorSubcoreMesh(core_axis_name="c", subcore_axis_name="s")

    @functools.partial(
        pl.kernel,
        mesh=mesh,
        compiler_params=pltpu.CompilerParams(
            needs_layout_passes=False, use_tc_tiling_on_sc=False),
        out_type=jax.ShapeDtypeStruct((E,), jnp.float32),
        scratch_types=[
            pltpu.VMEM((E_PER_W,), jnp.int32),
            pltpu.VMEM((E_PER_W,), jnp.int32),
            pltpu.VMEM((CHUNK, DW), jnp.int32),
            pltpu.VMEM((CHUNK, DW), jnp.int32),
            pltpu.VMEM((CHUNK, DW), jnp.int32),
            pltpu.VMEM((CHUNK, DW), jnp.int32),
            pltpu.VMEM((CHUNK, DW), jnp.int32),
            pltpu.VMEM((CHUNK, DW), jnp.int32),
            pltpu.VMEM((E_PER_W,), jnp.float32),
            pltpu.VMEM_SHARED((N_NODES, DW), jnp.int32),
            pltpu.SemaphoreType.DMA,
            pltpu.SemaphoreType.DMA,
            pltpu.SemaphoreType.DMA,
            pltpu.SemaphoreType.DMA,
            pltpu.SemaphoreType.DMA,
            pltpu.SemaphoreType.DMA,
        ],
    )
    def k(h_hbm, src_hbm, dst_hbm, out_hbm,
          src_v, dst_v, bs0, bs1, bs2, bd0, bd1, bd2, out_v, h_sh,
          sems0, sems1, sems2, semd0, semd1, semd2):
        wid = lax.axis_index("s") * NUM_CORES + lax.axis_index("c")
        base = wid * E_PER_W

        @pl.when(lax.axis_index("s") == 0)
        def _():
            pltpu.sync_copy(h_hbm, h_sh)

        pltpu.sync_copy(src_hbm.at[pl.ds(base, E_PER_W)], src_v)
        pltpu.sync_copy(dst_hbm.at[pl.ds(base, E_PER_W)], dst_v)
        plsc.subcore_barrier()

        bufs = ((bs0, bd0, sems0, semd0), (bs1, bd1, sems1, semd1),
                (bs2, bd2, sems2, semd2))

        def start(c, b):
            bs, bd, ssem, dsem = bufs[b]
            pltpu.async_copy(h_sh.at[src_v.at[pl.ds(c * CHUNK, CHUNK)]], bs, ssem)
            pltpu.async_copy(h_hbm.at[dst_v.at[pl.ds(c * CHUNK, CHUNK)]], bd, dsem)

        def wait(c, b):
            bs, bd, ssem, dsem = bufs[b]
            pltpu.make_async_copy(h_sh.at[src_v.at[pl.ds(c * CHUNK, CHUNK)]], bs, ssem).wait()
            pltpu.make_async_copy(h_hbm.at[dst_v.at[pl.ds(c * CHUNK, CHUNK)]], bd, dsem).wait()

        lane = lax.iota(jnp.int32, 16)
        bitrev4 = (0, 8, 4, 12, 2, 10, 6, 14, 1, 9, 5, 13, 3, 11, 7, 15)
        perms = {k: (lane ^ k).astype(jnp.int32) for k in (8, 4, 2, 1)}
        masks = {k: (lane & k) == 0 for k in (8, 4, 2, 1)}

        def fold(v, k):
            return v + jnp.take_along_axis(v, perms[k], axis=0)

        def edge_dot(bs, bd, e):
            terms = []
            for g in range(4):
                sb = plsc.bitcast(bs[e, pl.ds(g * 16, 16)], jnp.bfloat16)
                db = plsc.bitcast(bd[e, pl.ds(g * 16, 16)], jnp.bfloat16)
                lo, hi = plsc.unpack(sb * db, format=plsc.PackFormat.INTERLEAVED)
                terms += [lo, hi]
            t = [terms[2 * j] + terms[2 * j + 1] for j in range(4)]
            u = [t[0] + t[1], t[2] + t[3]]
            return u[0] + u[1]

        def compute(c, b):
            bs, bd = bufs[b][0], bufs[b][1]

            def group_body(gi, _):
                eb = gi * 16
                vecs = [edge_dot(bs, bd, eb + bitrev4[i]) for i in range(16)]
                for k in (8, 4, 2, 1):
                    vecs = [jnp.where(masks[k], fold(vecs[2 * j], k),
                                      fold(vecs[2 * j + 1], k))
                            for j in range(len(vecs) // 2)]
                out_v[pl.ds(c * CHUNK + eb, 16)] = vecs[0]
                return 0

            lax.fori_loop(0, CHUNK // 16, group_body, 0)

        for b in range(NRING):
            start(b, b)

        def ring_body(g, _):
            for b in range(NRING):
                c = NRING * g + b
                wait(c, b)
                compute(c, b)

                @pl.when(c + NRING < NCHUNK)
                def _():
                    start(c + NRING, b)
            return 0

        lax.fori_loop(0, NCHUNK // NRING, ring_body, 0)
        for c in range(NCHUNK - NCHUNK % NRING, NCHUNK):
            wait(c, c % NRING)
            compute(c, c % NRING)

        pltpu.sync_copy(out_v, out_hbm.at[pl.ds(base, E_PER_W)])

    return k(hp, src, dst)


def kernel(x, edge_index):
    ei = edge_index.astype(jnp.int32)
    hp, src, dst = _prep(x, ei)
    score = _sc_edge_dot(hp, src, dst)
    return score.reshape(E, 1)


# speedup vs baseline: 1.0147x; 1.0147x over previous
import functools

import jax
import jax.numpy as jnp
from jax import lax
from jax.experimental import pallas as pl
from jax.experimental.pallas import tpu as pltpu
from jax.experimental.pallas import tpu_sc as plsc

N_NODES = 10000
D = 128
DW = D // 2
E = 320000
NUM_CORES = 2
NUM_SUBCORES = 16
NW = NUM_CORES * NUM_SUBCORES
E_PER_W = E // NW
CHUNK = 80
NCHUNK = E_PER_W // CHUNK
NRING = 3


def _prep_body(x_ref, e_ref, h_ref, s_ref, d_ref):
    xv = x_ref[...]
    ss = jnp.sum(xv * xv, axis=-1, keepdims=True)
    nrm = jnp.sqrt(ss)
    h = xv / jnp.maximum(nrm, 1e-12)
    lo = lax.bitcast_convert_type(
        h[:, :DW].astype(jnp.bfloat16), jnp.uint16).astype(jnp.int32)
    hi = lax.bitcast_convert_type(
        h[:, DW:].astype(jnp.bfloat16), jnp.uint16).astype(jnp.int32)
    h_ref[...] = (hi << 16) | lo
    s_ref[...] = e_ref[0, :]
    d_ref[...] = e_ref[1, :]


def _prep(x, edges):
    return pl.pallas_call(
        _prep_body,
        out_shape=[
            jax.ShapeDtypeStruct((N_NODES, DW), jnp.int32),
            jax.ShapeDtypeStruct((E,), jnp.int32),
            jax.ShapeDtypeStruct((E,), jnp.int32),
        ],
    )(x, edges)


def _sc_edge_dot(hp, src, dst):
    mesh = plsc.VectorSubcoreMesh(core_axis_name="c", subcore_axis_name="s")

    @functools.partial(
        pl.kernel,
        mesh=mesh,
        compiler_params=pltpu.CompilerParams(
            needs_layout_passes=False, use_tc_tiling_on_sc=False),
        out_type=jax.ShapeDtypeStruct((E,), jnp.float32),
        scratch_types=[
            pltpu.VMEM((E_PER_W,), jnp.int32),
            pltpu.VMEM((E_PER_W,), jnp.int32),
            pltpu.VMEM((CHUNK, DW), jnp.int32),
            pltpu.VMEM((CHUNK, DW), jnp.int32),
            pltpu.VMEM((CHUNK, DW), jnp.int32),
            pltpu.VMEM((CHUNK, DW), jnp.int32),
            pltpu.VMEM((CHUNK, DW), jnp.int32),
            pltpu.VMEM((CHUNK, DW), jnp.int32),
            pltpu.VMEM((E_PER_W,), jnp.float32),
            pltpu.VMEM_SHARED((N_NODES, DW), jnp.int32),
            pltpu.SemaphoreType.DMA,
            pltpu.SemaphoreType.DMA,
            pltpu.SemaphoreType.DMA,
        ],
    )
    def k(h_hbm, src_hbm, dst_hbm, out_hbm,
          src_v, dst_v, bs0, bs1, bs2, bd0, bd1, bd2, out_v, h_sh,
          sem0, sem1, sem2):
        wid = lax.axis_index("s") * NUM_CORES + lax.axis_index("c")
        base = wid * E_PER_W

        @pl.when(lax.axis_index("s") == 0)
        def _():
            pltpu.sync_copy(h_hbm, h_sh)

        pltpu.sync_copy(src_hbm.at[pl.ds(base, E_PER_W)], src_v)
        pltpu.sync_copy(dst_hbm.at[pl.ds(base, E_PER_W)], dst_v)
        plsc.subcore_barrier()

        bufs = ((bs0, bd0, sem0), (bs1, bd1, sem1), (bs2, bd2, sem2))

        def start(c, b):
            bs, bd, sem = bufs[b]
            pltpu.async_copy(h_sh.at[src_v.at[pl.ds(c * CHUNK, CHUNK)]], bs, sem)
            pltpu.async_copy(h_sh.at[dst_v.at[pl.ds(c * CHUNK, CHUNK)]], bd, sem)

        def wait(c, b):
            bs, bd, sem = bufs[b]
            pltpu.make_async_copy(h_sh.at[src_v.at[pl.ds(c * CHUNK, CHUNK)]], bs, sem).wait()
            pltpu.make_async_copy(h_sh.at[dst_v.at[pl.ds(c * CHUNK, CHUNK)]], bd, sem).wait()

        lane = lax.iota(jnp.int32, 16)
        bitrev4 = (0, 8, 4, 12, 2, 10, 6, 14, 1, 9, 5, 13, 3, 11, 7, 15)
        perms = {k: (lane ^ k).astype(jnp.int32) for k in (8, 4, 2, 1)}
        masks = {k: (lane & k) == 0 for k in (8, 4, 2, 1)}

        def fold(v, k):
            return v + jnp.take_along_axis(v, perms[k], axis=0)

        def edge_dot(bs, bd, e):
            terms = []
            for g in range(4):
                sb = plsc.bitcast(bs[e, pl.ds(g * 16, 16)], jnp.bfloat16)
                db = plsc.bitcast(bd[e, pl.ds(g * 16, 16)], jnp.bfloat16)
                lo, hi = plsc.unpack(sb * db, format=plsc.PackFormat.INTERLEAVED)
                terms += [lo, hi]
            t = [terms[2 * j] + terms[2 * j + 1] for j in range(4)]
            u = [t[0] + t[1], t[2] + t[3]]
            return u[0] + u[1]

        def compute(c, b):
            bs, bd, _ = bufs[b]

            def group_body(gi, _):
                eb = gi * 16
                vecs = [edge_dot(bs, bd, eb + bitrev4[i]) for i in range(16)]
                for k in (8, 4, 2, 1):
                    vecs = [jnp.where(masks[k], fold(vecs[2 * j], k),
                                      fold(vecs[2 * j + 1], k))
                            for j in range(len(vecs) // 2)]
                out_v[pl.ds(c * CHUNK + eb, 16)] = vecs[0]
                return 0

            lax.fori_loop(0, CHUNK // 16, group_body, 0)

        for b in range(NRING):
            start(b, b)

        def ring_body(g, _):
            for b in range(NRING):
                c = NRING * g + b
                wait(c, b)
                compute(c, b)

                @pl.when(c + NRING < NCHUNK)
                def _():
                    start(c + NRING, b)
            return 0

        lax.fori_loop(0, NCHUNK // NRING, ring_body, 0)
        for c in range(NCHUNK - NCHUNK % NRING, NCHUNK):
            wait(c, c % NRING)
            compute(c, c % NRING)

        pltpu.sync_copy(out_v, out_hbm.at[pl.ds(base, E_PER_W)])

    return k(hp, src, dst)


def kernel(x, edge_index):
    ei = edge_index.astype(jnp.int32)
    hp, src, dst = _prep(x, ei)
    score = _sc_edge_dot(hp, src, dst)
    return score.reshape(E, 1)
